# Initial kernel scaffold; baseline (speedup 1.0000x reference)
import jax
import jax.numpy as jnp
from jax.experimental import pallas as pl


def kernel(x, edge_index, W1, b1, W2, b2):
    raise NotImplementedError("write your pallas kernel here")


# speedup vs baseline: 21.5147x; 21.5147x over previous
import functools

import jax
import jax.numpy as jnp
from jax import lax
from jax.experimental import pallas as pl
from jax.experimental.pallas import tpu as pltpu
from jax.experimental.pallas import tpu_sc as plsc

N_NODES = 10000
NUM_FEATURES = 128
HIDDEN_DIM = 64
NUM_CLASSES = 16

NC = 2
NS = 16
NW = NC * NS
K = 128
CHUNKS = 80
E_PAD = NW * K * CHUNKS
N_PAD = 10240
RPT = N_PAD // NS


def _sc_mesh():
    return plsc.VectorSubcoreMesh(core_axis_name="c", subcore_axis_name="s")


_SC_PARAMS = pltpu.CompilerParams(use_tc_tiling_on_sc=False)


DEG_W = 16


def _deg_body(dst_hbm, ones_hbm, z_hbm, out_hbm, dst_v, ones_v, acc):
    cid = lax.axis_index("c")
    sid = lax.axis_index("s")
    wid = sid * NC + cid
    pltpu.sync_copy(dst_hbm.at[wid], dst_v)
    pltpu.sync_copy(ones_hbm, ones_v)
    sl = pl.ds(sid * RPT, RPT)
    pltpu.sync_copy(z_hbm, acc.at[sl])
    plsc.subcore_barrier()

    def step(j, carry):
        pltpu.sync_copy(ones_v, acc.at[dst_v.at[j]], add=True)
        return carry

    lax.fori_loop(0, CHUNKS, step, 0)
    plsc.subcore_barrier()
    pltpu.sync_copy(acc.at[sl], out_hbm.at[cid, sl])


@jax.jit
def _sc_degree(dst3, ones, zeros1):
    return pl.kernel(
        _deg_body,
        out_type=jax.ShapeDtypeStruct((NC, N_PAD, DEG_W), jnp.float32),
        mesh=_sc_mesh(),
        scratch_types=[
            pltpu.VMEM((CHUNKS, K), jnp.int32),
            pltpu.VMEM((K, DEG_W), jnp.float32),
            pltpu.VMEM_SHARED((N_PAD, DEG_W), jnp.float32),
        ],
        compiler_params=_SC_PARAMS,
    )(dst3, ones, zeros1)


def _agg_body(h_hbm, src_hbm, dst_hbm, z_hbm, out_hbm,
              src_v, dst_v, rows0, rows1, acc, sem0, sem1):
    cid = lax.axis_index("c")
    sid = lax.axis_index("s")
    wid = sid * NC + cid
    pltpu.sync_copy(src_hbm.at[wid], src_v)
    pltpu.sync_copy(dst_hbm.at[wid], dst_v)
    sl = pl.ds(sid * RPT, RPT)
    pltpu.sync_copy(z_hbm, acc.at[sl])
    plsc.subcore_barrier()

    pltpu.async_copy(h_hbm.at[src_v.at[0]], rows0, sem0)
    pltpu.async_copy(h_hbm.at[src_v.at[1]], rows1, sem1)

    def step(t, carry):
        j0 = t * 2
        for b, (rv, sb) in enumerate(((rows0, sem0), (rows1, sem1))):
            j = j0 + b
            pltpu.make_async_copy(h_hbm.at[src_v.at[j]], rv, sb).wait()
            pltpu.sync_copy(rv, acc.at[dst_v.at[j]], add=True)
            jn = jnp.minimum(j + 2, CHUNKS - 1)
            pltpu.async_copy(h_hbm.at[src_v.at[jn]], rv, sb)
        return carry

    lax.fori_loop(0, CHUNKS // 2, step, 0)
    pltpu.make_async_copy(h_hbm.at[src_v.at[0]], rows0, sem0).wait()
    pltpu.make_async_copy(h_hbm.at[src_v.at[0]], rows1, sem1).wait()
    plsc.subcore_barrier()
    pltpu.sync_copy(acc.at[sl], out_hbm.at[cid, sl])


@functools.partial(jax.jit, static_argnames=("d",))
def _sc_aggregate(h, src3, dst3, zeros, d):
    return pl.kernel(
        _agg_body,
        out_type=jax.ShapeDtypeStruct((NC, N_PAD, d), jnp.float32),
        mesh=_sc_mesh(),
        scratch_types=[
            pltpu.VMEM((CHUNKS, K), jnp.int32),
            pltpu.VMEM((CHUNKS, K), jnp.int32),
            pltpu.VMEM((K, d), jnp.float32),
            pltpu.VMEM((K, d), jnp.float32),
            pltpu.VMEM_SHARED((N_PAD, d), jnp.float32),
            pltpu.SemaphoreType.DMA,
            pltpu.SemaphoreType.DMA,
        ],
        compiler_params=_SC_PARAMS,
    )(h, src3, dst3, zeros)


_BT = 512


def _l1_body(x_ref, w_ref, degp_ref, h_ref, dinv_ref):
    deg = degp_ref[0, :, 0:1] + degp_ref[1, :, 0:1] + 1.0
    dinv = lax.rsqrt(deg)
    h_ref[...] = jnp.dot(x_ref[...], w_ref[...],
                         preferred_element_type=jnp.float32) * dinv
    dinv_ref[...] = dinv


def _tc_layer1(x_pad, w1, degp):
    grid = (N_PAD // _BT,)
    return pl.pallas_call(
        _l1_body,
        grid=grid,
        in_specs=[
            pl.BlockSpec((_BT, NUM_FEATURES), lambda i: (i, 0)),
            pl.BlockSpec((NUM_FEATURES, HIDDEN_DIM), lambda i: (0, 0)),
            pl.BlockSpec((NC, _BT, DEG_W), lambda i: (0, i, 0)),
        ],
        out_specs=[
            pl.BlockSpec((_BT, HIDDEN_DIM), lambda i: (i, 0)),
            pl.BlockSpec((_BT, 1), lambda i: (i, 0)),
        ],
        out_shape=[
            jax.ShapeDtypeStruct((N_PAD, HIDDEN_DIM), jnp.float32),
            jax.ShapeDtypeStruct((N_PAD, 1), jnp.float32),
        ],
    )(x_pad, w1, degp)


def _mid_body(part_ref, h1s_ref, dinv_ref, w2_ref, b1_ref, h2s_ref):
    s1 = part_ref[0] + part_ref[1] + h1s_ref[...]
    dinv = dinv_ref[...]
    g1 = jnp.maximum(s1 * dinv + b1_ref[...], 0.0)
    h2s_ref[...] = jnp.dot(g1, w2_ref[...],
                           preferred_element_type=jnp.float32) * dinv


def _tc_mid(part1, h1s, dinv, w2, b1r):
    grid = (N_PAD // _BT,)
    return pl.pallas_call(
        _mid_body,
        grid=grid,
        in_specs=[
            pl.BlockSpec((NC, _BT, HIDDEN_DIM), lambda i: (0, i, 0)),
            pl.BlockSpec((_BT, HIDDEN_DIM), lambda i: (i, 0)),
            pl.BlockSpec((_BT, 1), lambda i: (i, 0)),
            pl.BlockSpec((HIDDEN_DIM, NUM_CLASSES), lambda i: (0, 0)),
            pl.BlockSpec((1, HIDDEN_DIM), lambda i: (0, 0)),
        ],
        out_specs=pl.BlockSpec((_BT, NUM_CLASSES), lambda i: (i, 0)),
        out_shape=jax.ShapeDtypeStruct((N_PAD, NUM_CLASSES), jnp.float32),
    )(part1, h1s, dinv, w2, b1r)


def _fin_body(part_ref, h2s_ref, dinv_ref, b2_ref, o_ref):
    s2 = part_ref[0] + part_ref[1] + h2s_ref[...]
    o_ref[...] = jax.nn.sigmoid(s2 * dinv_ref[...] + b2_ref[...])


def _tc_final(part2, h2s, dinv, b2r):
    grid = (N_PAD // _BT,)
    return pl.pallas_call(
        _fin_body,
        grid=grid,
        in_specs=[
            pl.BlockSpec((NC, _BT, NUM_CLASSES), lambda i: (0, i, 0)),
            pl.BlockSpec((_BT, NUM_CLASSES), lambda i: (i, 0)),
            pl.BlockSpec((_BT, 1), lambda i: (i, 0)),
            pl.BlockSpec((1, NUM_CLASSES), lambda i: (0, 0)),
        ],
        out_specs=pl.BlockSpec((_BT, NUM_CLASSES), lambda i: (i, 0)),
        out_shape=jax.ShapeDtypeStruct((N_PAD, NUM_CLASSES), jnp.float32),
    )(part2, h2s, dinv, b2r)


def kernel(x, edge_index, W1, b1, W2, b2):
    E = edge_index.shape[1]
    pad = E_PAD - E
    ei = edge_index.astype(jnp.int32)
    src3 = jnp.concatenate(
        [ei[0], jnp.full((pad,), N_NODES, jnp.int32)]).reshape(NW, CHUNKS, K)
    dst3 = jnp.concatenate(
        [ei[1], jnp.full((pad,), N_NODES, jnp.int32)]).reshape(NW, CHUNKS, K)
    x_pad = jnp.pad(x, ((0, N_PAD - N_NODES), (0, 0)))
    ones = jnp.ones((K, DEG_W), jnp.float32)
    z64 = jnp.zeros((RPT, HIDDEN_DIM), jnp.float32)
    z16 = jnp.zeros((RPT, NUM_CLASSES), jnp.float32)

    degp = _sc_degree(dst3, ones, z16)
    h1s, dinv = _tc_layer1(x_pad, W1, degp)
    part1 = _sc_aggregate(h1s, src3, dst3, z64, HIDDEN_DIM)
    h2s = _tc_mid(part1, h1s, dinv, W2, b1.reshape(1, HIDDEN_DIM))
    part2 = _sc_aggregate(h2s, src3, dst3, z16, NUM_CLASSES)
    out = _tc_final(part2, h2s, dinv, b2.reshape(1, NUM_CLASSES))
    return out[:N_NODES]
